# Initial kernel scaffold; baseline (speedup 1.0000x reference)
import functools

import jax
import jax.numpy as jnp
import numpy as np
from jax import lax
from jax.experimental import pallas as pl
from jax.experimental.pallas import tpu as pltpu
from jax.experimental.pallas import tpu_sc as plsc

H = 8
D = 16
HD = H * D
NA = 1
CLAMP = 5.0
N_NODES = 10000
N_EDGES = 320000
F = 128

NC = 2
NS = 16
NW = NC * NS
CHUNK = N_EDGES // NW
BLK = 80
NBLK = CHUNK // BLK
NGRP = BLK // 16
ROWS_PER_TILE = N_NODES // NS

RB = 1000


def _dense_body(x_ref, wqt_ref, wkt_ref, wvt_ref, bq_ref, bk_ref, bv_ref,
                ang_ref, s_ref, rep_ref, rotm_ref, q_out, k_out, v_out):
    xb = x_ref[...]
    q = jnp.dot(xb, wqt_ref[...], preferred_element_type=jnp.float32) + bq_ref[...]
    k = jnp.dot(xb, wkt_ref[...], preferred_element_type=jnp.float32) + bk_ref[...]
    v = jnp.dot(xb, wvt_ref[...], preferred_element_type=jnp.float32) + bv_ref[...]

    srow = s_ref[...]
    m = jnp.max(srow, axis=1, keepdims=True)
    e = jnp.exp(srow - m)
    ssm = e / jnp.sum(e, axis=1, keepdims=True)
    srep = jnp.dot(ssm, rep_ref[...], preferred_element_type=jnp.float32)

    theta = jnp.dot(ang_ref[...], srep, preferred_element_type=jnp.float32)
    c = jnp.cos(theta)
    s = jnp.sin(theta)
    rotm = rotm_ref[...]
    q_out[...] = q * c + jnp.dot(q, rotm, preferred_element_type=jnp.float32) * s
    k_out[...] = k * c + jnp.dot(k, rotm, preferred_element_type=jnp.float32) * s
    v_out[...] = v


def _dense_call(x, ang, wqt, wkt, wvt, bq, bk, bv, s, rep, rotm):
    n = x.shape[0]
    grid = (n // RB,)
    full = lambda shp: pl.BlockSpec(shp, lambda i: (0,) * len(shp))
    return pl.pallas_call(
        _dense_body,
        grid=grid,
        in_specs=[
            pl.BlockSpec((RB, F), lambda i: (i, 0)),
            full((F, HD)), full((F, HD)), full((F, HD)),
            full((1, HD)), full((1, HD)), full((1, HD)),
            pl.BlockSpec((RB, NA), lambda i: (i, 0)),
            full((NA, HD // 2)), full((HD // 2, HD)), full((HD, HD)),
        ],
        out_specs=[pl.BlockSpec((RB, HD), lambda i: (i, 0))] * 3,
        out_shape=[jax.ShapeDtypeStruct((n, HD), jnp.float32)] * 3,
    )(x, wqt, wkt, wvt, bq, bk, bv, ang, s, rep, rotm)


def _sc_body(k_hbm, q_hbm, v_hbm, eidx_hbm, z128_hbm, z16_hbm,
             numer_out, denom_out,
             sidx, didx, kbuf, qbuf, vbuf, wbuf, sh_num, sh_den,
             sem0, sem1, sem2):
    cid = lax.axis_index("c")
    sid = lax.axis_index("s")
    wid = sid * NC + cid
    base = wid * CHUNK
    tb = sid * ROWS_PER_TILE

    pltpu.sync_copy(z128_hbm, sh_num.at[pl.ds(tb, ROWS_PER_TILE)])
    pltpu.sync_copy(z16_hbm, sh_den.at[pl.ds(tb, ROWS_PER_TILE)])

    @pl.loop(0, BLK)
    def _zero_w(i):
        wbuf[i] = jnp.zeros((16,), jnp.float32)

    plsc.subcore_barrier()

    @pl.loop(0, NBLK)
    def _block(b):
        eb = base + b * BLK
        pltpu.sync_copy(eidx_hbm.at[0, pl.ds(eb, BLK)], sidx.at[0])
        pltpu.sync_copy(eidx_hbm.at[1, pl.ds(eb, BLK)], didx.at[0])
        cp_k = pltpu.async_copy(k_hbm.at[sidx.at[0]], kbuf, sem0)
        cp_q = pltpu.async_copy(q_hbm.at[didx.at[0]], qbuf, sem1)
        cp_v = pltpu.async_copy(v_hbm.at[sidx.at[0]], vbuf, sem2)
        cp_k.wait()
        cp_q.wait()
        cp_v.wait()

        @pl.loop(0, NGRP)
        def _group(g):
            j0 = g * 16
            eidx = j0 + lax.iota(jnp.int32, 16)
            inv_sqrt_d = 1.0 / (D ** 0.5)
            for h in range(H):
                col0 = h * D
                acc = jnp.zeros((16,), jnp.float32)
                for dd in range(D):
                    cvec = jnp.full((16,), col0 + dd, jnp.int32)
                    kc = plsc.load_gather(kbuf, [eidx, cvec])
                    qc = plsc.load_gather(qbuf, [eidx, cvec])
                    acc = acc + kc * qc
                sc = jnp.clip(acc * inv_sqrt_d, -CLAMP, CLAMP)
                wh = jnp.exp(sc)
                plsc.store_scatter(wbuf, [eidx, jnp.full((16,), h, jnp.int32)], wh)
            for j in range(16):
                row = j0 + j
                rowv = jnp.broadcast_to(row, (16,)).astype(jnp.int32)
                for h in range(H):
                    wb = plsc.load_gather(wbuf, [rowv, jnp.full((16,), h, jnp.int32)])
                    vrow = vbuf[row, pl.ds(h * D, 16)]
                    vbuf[row, pl.ds(h * D, 16)] = vrow * wb

        pltpu.sync_copy(vbuf, sh_num.at[didx.at[0]], add=True)
        pltpu.sync_copy(wbuf, sh_den.at[didx.at[0]], add=True)

    plsc.subcore_barrier()

    pltpu.sync_copy(sh_num.at[pl.ds(tb, ROWS_PER_TILE)],
                    numer_out.at[cid, pl.ds(tb, ROWS_PER_TILE)])
    pltpu.sync_copy(sh_den.at[pl.ds(tb, ROWS_PER_TILE)],
                    denom_out.at[cid, pl.ds(tb, ROWS_PER_TILE)])


def _sc_call(k_rot, q_rot, v_h, edge_index, z128, z16):
    mesh = plsc.VectorSubcoreMesh(core_axis_name="c", subcore_axis_name="s")
    kern = functools.partial(
        pl.kernel,
        out_type=[
            jax.ShapeDtypeStruct((NC, N_NODES, HD), jnp.float32),
            jax.ShapeDtypeStruct((NC, N_NODES, 16), jnp.float32),
        ],
        mesh=mesh,
        scratch_types=[
            pltpu.VMEM((1, BLK), jnp.int32),
            pltpu.VMEM((1, BLK), jnp.int32),
            pltpu.VMEM((BLK, HD), jnp.float32),
            pltpu.VMEM((BLK, HD), jnp.float32),
            pltpu.VMEM((BLK, HD), jnp.float32),
            pltpu.VMEM((BLK, 16), jnp.float32),
            pltpu.VMEM_SHARED((N_NODES, HD), jnp.float32),
            pltpu.VMEM_SHARED((N_NODES, 16), jnp.float32),
            pltpu.SemaphoreType.DMA,
            pltpu.SemaphoreType.DMA,
            pltpu.SemaphoreType.DMA,
        ],
    )(_sc_body)
    return kern(k_rot, q_rot, v_h, edge_index, z128, z16)


def _combine_body(num_ref, den_ref, sel_ref, out_ref):
    nsum = num_ref[0] + num_ref[1]
    dsum = den_ref[0] + den_ref[1]
    drep = jnp.dot(dsum, sel_ref[...], preferred_element_type=jnp.float32)
    out_ref[...] = nsum / (drep + 1e-16)


def _combine_call(numer, denom, sel):
    grid = (N_NODES // RB,)
    return pl.pallas_call(
        _combine_body,
        grid=grid,
        in_specs=[
            pl.BlockSpec((NC, RB, HD), lambda i: (0, i, 0)),
            pl.BlockSpec((NC, RB, 16), lambda i: (0, i, 0)),
            pl.BlockSpec((16, HD), lambda i: (0, 0)),
        ],
        out_specs=pl.BlockSpec((RB, HD), lambda i: (i, 0)),
        out_shape=jax.ShapeDtypeStruct((N_NODES, HD), jnp.float32),
    )(numer, denom, sel)


def kernel(x, edge_index, node_rotation_angles, Wq, bq, Wk, bk, Wv, bv, S):
    n = x.shape[0]

    rotm = np.zeros((HD, HD), np.float32)
    for i in range(HD // 2):
        rotm[2 * i + 1, 2 * i] = -1.0
        rotm[2 * i, 2 * i + 1] = 1.0
    rep = np.zeros((HD // 2, HD), np.float32)
    for i in range(HD // 2):
        rep[i, 2 * i] = 1.0
        rep[i, 2 * i + 1] = 1.0
    sel = np.zeros((16, HD), np.float32)
    for h in range(H):
        sel[h, h * D:(h + 1) * D] = 1.0
    rotm = jnp.asarray(rotm)
    rep = jnp.asarray(rep)
    sel = jnp.asarray(sel)

    q_rot, k_rot, v_h = _dense_call(
        x, node_rotation_angles,
        Wq.T, Wk.T, Wv.T,
        bq.reshape(1, HD), bk.reshape(1, HD), bv.reshape(1, HD),
        S, rep, rotm)

    z128 = jnp.zeros((ROWS_PER_TILE, HD), jnp.float32)
    z16 = jnp.zeros((ROWS_PER_TILE, 16), jnp.float32)
    numer, denom = _sc_call(k_rot, q_rot, v_h, edge_index, z128, z16)

    wv = _combine_call(numer, denom, sel)
    return wv.reshape(n, H, D)


# speedup vs baseline: 17.5012x; 17.5012x over previous
import functools

import jax
import jax.numpy as jnp
import numpy as np
from jax import lax
from jax.experimental import pallas as pl
from jax.experimental.pallas import tpu as pltpu
from jax.experimental.pallas import tpu_sc as plsc

H = 8
D = 16
HD = H * D
NA = 1
CLAMP = 5.0
N_NODES = 10000
N_EDGES = 320000
F = 128

NC = 2
NS = 16
NW = NC * NS
CHUNK = N_EDGES // NW
BLK = 80
NBLK = CHUNK // BLK
NGRP = BLK // 16
NPAD = 10112
ROWS_PER_TILE = NPAD // NS
NDEN = NPAD // 16

RB = 1000


def _dense_body(x_ref, wqt_ref, wkt_ref, wvt_ref, bq_ref, bk_ref, bv_ref,
                ang_ref, s_ref, rep_ref, rotm_ref, q_out, k_out, v_out):
    xb = x_ref[...]
    q = jnp.dot(xb, wqt_ref[...], preferred_element_type=jnp.float32) + bq_ref[...]
    k = jnp.dot(xb, wkt_ref[...], preferred_element_type=jnp.float32) + bk_ref[...]
    v = jnp.dot(xb, wvt_ref[...], preferred_element_type=jnp.float32) + bv_ref[...]

    srow = s_ref[...]
    m = jnp.max(srow, axis=1, keepdims=True)
    e = jnp.exp(srow - m)
    ssm = e / jnp.sum(e, axis=1, keepdims=True)
    srep = jnp.dot(ssm, rep_ref[...], preferred_element_type=jnp.float32)

    theta = ang_ref[...] * srep
    c = jnp.cos(theta)
    s = jnp.sin(theta)
    rotm = rotm_ref[...]
    q_out[...] = q * c + jnp.dot(q, rotm, preferred_element_type=jnp.float32) * s
    k_out[...] = k * c + jnp.dot(k, rotm, preferred_element_type=jnp.float32) * s
    v_out[...] = v


def _dense_call(x, ang, wqt, wkt, wvt, bq, bk, bv, s, rep, rotm):
    n = x.shape[0]
    grid = (n // RB,)
    full = lambda shp: pl.BlockSpec(shp, lambda i: (0,) * len(shp))
    return pl.pallas_call(
        _dense_body,
        grid=grid,
        in_specs=[
            pl.BlockSpec((RB, F), lambda i: (i, 0)),
            full((F, HD)), full((F, HD)), full((F, HD)),
            full((1, HD)), full((1, HD)), full((1, HD)),
            pl.BlockSpec((RB, NA), lambda i: (i, 0)),
            full((NA, HD // 2)), full((HD // 2, HD)), full((HD, HD)),
        ],
        out_specs=[pl.BlockSpec((RB, HD), lambda i: (i, 0))] * 3,
        out_shape=[jax.ShapeDtypeStruct((n, HD), jnp.float32)] * 3,
    )(x, wqt, wkt, wvt, bq, bk, bv, ang, s, rep, rotm)


def _chunks(total, step):
    out = []
    r = 0
    while r < total:
        n = min(step, total - r)
        out.append((r, n))
        r += n
    return out


def _sc_body(k_hbm, q_hbm, v_hbm, src_hbm, dst_hbm,
             acc_out, den_out,
             sidx, didx, didx16, kbuf, qbuf, dbuf, wbuf, sh_acc, sh_den,
             sem0, sem1, sem2):
    cid = lax.axis_index("c")
    sid = lax.axis_index("s")
    wid = sid * NC + cid
    base = wid * CHUNK
    tb = sid * ROWS_PER_TILE

    @pl.loop(0, BLK)
    def _zero_bufs(i):
        wbuf[i] = jnp.zeros((16,), jnp.float32)
        for c in range(H):
            kbuf[i, pl.ds(c * 16, 16)] = jnp.zeros((16,), jnp.float32)
            dbuf[i, pl.ds(c * 16, 16)] = jnp.zeros((16,), jnp.float32)

    for r0, nr in _chunks(ROWS_PER_TILE, BLK):
        pltpu.sync_copy(kbuf.at[pl.ds(0, nr)], sh_acc.at[pl.ds(tb + r0, nr)])

    @pl.when(sid == 0)
    def _zero_den():
        for r0, nr in _chunks(NDEN, BLK):
            pltpu.sync_copy(dbuf.at[pl.ds(0, nr)], sh_den.at[pl.ds(r0, nr)])

    plsc.subcore_barrier()

    @pl.loop(0, NBLK)
    def _block(b):
        eb = base + b * BLK
        pltpu.sync_copy(src_hbm.at[pl.ds(eb, BLK)], sidx.at[0])
        pltpu.sync_copy(dst_hbm.at[pl.ds(eb, BLK)], didx.at[0])
        cp_k = pltpu.async_copy(k_hbm.at[sidx.at[0]], kbuf, sem0)
        cp_q = pltpu.async_copy(q_hbm.at[didx.at[0]], qbuf, sem1)
        cp_k.wait()
        cp_q.wait()

        @pl.loop(0, NGRP)
        def _group(g):
            j0 = g * 16
            eidx = j0 + lax.iota(jnp.int32, 16)
            inv_sqrt_d = 1.0 / (D ** 0.5)
            for h in range(H):
                col0 = h * D
                acc = jnp.zeros((16,), jnp.float32)
                for dd in range(D):
                    cvec = jnp.full((16,), col0 + dd, jnp.int32)
                    kc = plsc.load_gather(kbuf, [eidx, cvec])
                    qc = plsc.load_gather(qbuf, [eidx, cvec])
                    acc = acc + kc * qc
                sc = jnp.clip(acc * inv_sqrt_d, -CLAMP, CLAMP)
                wh = jnp.exp(sc)
                plsc.store_scatter(wbuf, [eidx, jnp.full((16,), h, jnp.int32)], wh)
            dvec = didx[0, pl.ds(j0, 16)]
            didx16[0, pl.ds(j0, 16)] = lax.shift_right_logical(dvec, 4)
            colbase = lax.shift_left(dvec & 15, 3)
            for h in range(H):
                wh = plsc.load_gather(wbuf, [eidx, jnp.full((16,), h, jnp.int32)])
                plsc.store_scatter(dbuf, [eidx, colbase + h], wh)

        pltpu.async_copy(v_hbm.at[sidx.at[0]], kbuf, sem2).wait()

        @pl.loop(0, BLK)
        def _scale(row):
            rowv = jnp.broadcast_to(row, (16,)).astype(jnp.int32)
            for h in range(H):
                wb = plsc.load_gather(wbuf, [rowv, jnp.full((16,), h, jnp.int32)])
                vrow = kbuf[row, pl.ds(h * D, 16)]
                kbuf[row, pl.ds(h * D, 16)] = vrow * wb

        pltpu.sync_copy(kbuf, sh_acc.at[didx.at[0]], add=True)
        pltpu.sync_copy(dbuf, sh_den.at[didx16.at[0]], add=True)

        @pl.loop(0, BLK)
        def _zero_d(i):
            for c in range(H):
                dbuf[i, pl.ds(c * 16, 16)] = jnp.zeros((16,), jnp.float32)

    plsc.subcore_barrier()

    for r0, nr in _chunks(ROWS_PER_TILE, BLK):
        pltpu.sync_copy(sh_acc.at[pl.ds(tb + r0, nr)], kbuf.at[pl.ds(0, nr)])
        pltpu.sync_copy(kbuf.at[pl.ds(0, nr)],
                        acc_out.at[cid, pl.ds(tb + r0, nr)])

    @pl.when(sid == 0)
    def _dump_den():
        for r0, nr in _chunks(NDEN, BLK):
            pltpu.sync_copy(sh_den.at[pl.ds(r0, nr)], dbuf.at[pl.ds(0, nr)])
            pltpu.sync_copy(dbuf.at[pl.ds(0, nr)],
                            den_out.at[cid, pl.ds(r0, nr)])


def _sc_call(k_rot, q_rot, v_h, src, dst):
    mesh = plsc.VectorSubcoreMesh(core_axis_name="c", subcore_axis_name="s")
    kern = functools.partial(
        pl.kernel,
        out_type=[
            jax.ShapeDtypeStruct((NC, NPAD, HD), jnp.float32),
            jax.ShapeDtypeStruct((NC, NDEN, HD), jnp.float32),
        ],
        mesh=mesh,
        compiler_params=pltpu.CompilerParams(needs_layout_passes=False),
        scratch_types=[
            pltpu.VMEM((1, BLK), jnp.int32),
            pltpu.VMEM((1, BLK), jnp.int32),
            pltpu.VMEM((1, BLK), jnp.int32),
            pltpu.VMEM((BLK, HD), jnp.float32),
            pltpu.VMEM((BLK, HD), jnp.float32),
            pltpu.VMEM((BLK, HD), jnp.float32),
            pltpu.VMEM((BLK, 16), jnp.float32),
            pltpu.VMEM_SHARED((NPAD, HD), jnp.float32),
            pltpu.VMEM_SHARED((NDEN, HD), jnp.float32),
            pltpu.SemaphoreType.DMA,
            pltpu.SemaphoreType.DMA,
            pltpu.SemaphoreType.DMA,
        ],
    )(_sc_body)
    return kern(k_rot, q_rot, v_h, src, dst)


def _combine_body(acc_ref, den_ref, sel_ref, out_ref):
    nsum = acc_ref[0] + acc_ref[1]
    den8 = den_ref[0] + den_ref[1]
    drep = jnp.dot(den8, sel_ref[...], preferred_element_type=jnp.float32)
    out_ref[...] = nsum / (drep + 1e-16)


def _combine_call(acc, den, sel):
    grid = (N_NODES // RB,)
    return pl.pallas_call(
        _combine_body,
        grid=grid,
        in_specs=[
            pl.BlockSpec((NC, RB, HD), lambda i: (0, i, 0)),
            pl.BlockSpec((NC, RB, H), lambda i: (0, i, 0)),
            pl.BlockSpec((H, HD), lambda i: (0, 0)),
        ],
        out_specs=pl.BlockSpec((RB, HD), lambda i: (i, 0)),
        out_shape=jax.ShapeDtypeStruct((N_NODES, HD), jnp.float32),
    )(acc, den, sel)


def kernel(x, edge_index, node_rotation_angles, Wq, bq, Wk, bk, Wv, bv, S):
    n = x.shape[0]

    rotm = np.zeros((HD, HD), np.float32)
    for h in range(H):
        b = h * D
        for j in range(D // 2):
            rotm[b + 2 * j + 1, b + j] = -1.0
            rotm[b + 2 * j, b + D // 2 + j] = 1.0
    rep = np.zeros((HD // 2, HD), np.float32)
    for i in range(HD // 2):
        rep[i, 2 * i] = 1.0
        rep[i, 2 * i + 1] = 1.0
    sel = np.zeros((H, HD), np.float32)
    for h in range(H):
        sel[h, h * D:(h + 1) * D] = 1.0
    rotm = jnp.asarray(rotm)
    rep = jnp.asarray(rep)
    sel = jnp.asarray(sel)

    q_rot, k_rot, v_h = _dense_call(
        x, node_rotation_angles,
        Wq.T, Wk.T, Wv.T,
        bq.reshape(1, HD), bk.reshape(1, HD), bv.reshape(1, HD),
        S, rep, rotm)

    src = edge_index[0]
    dst = edge_index[1]
    acc, den = _sc_call(k_rot, q_rot, v_h, src, dst)

    den_nodes = den.reshape(NC, NPAD, H)
    wv = _combine_call(acc, den_nodes, sel)
    return wv.reshape(n, H, D)


# speedup vs baseline: 18.7040x; 1.0687x over previous
import functools

import jax
import jax.numpy as jnp
import numpy as np
from jax import lax
from jax.experimental import pallas as pl
from jax.experimental.pallas import tpu as pltpu
from jax.experimental.pallas import tpu_sc as plsc

H = 8
D = 16
HD = H * D
NA = 1
CLAMP = 5.0
N_NODES = 10000
N_EDGES = 320000
F = 128

NC = 2
NS = 16
NW = NC * NS
CHUNK = N_EDGES // NW
BLK = 80
NBLK = CHUNK // BLK
SB = 5
NSUP = NBLK // SB
NGRP = BLK // 16
NPAD = 10112
ROWS_PER_TILE = NPAD // NS
NDEN = NPAD // 16

RB = 1000


def _dense_body(x_ref, wqt_ref, wkt_ref, wvt_ref, bq_ref, bk_ref, bv_ref,
                ang_ref, s_ref, rep_ref, rotm_ref, q_out, k_out, v_out):
    xb = x_ref[...]
    q = jnp.dot(xb, wqt_ref[...], preferred_element_type=jnp.float32) + bq_ref[...]
    k = jnp.dot(xb, wkt_ref[...], preferred_element_type=jnp.float32) + bk_ref[...]
    v = jnp.dot(xb, wvt_ref[...], preferred_element_type=jnp.float32) + bv_ref[...]

    srow = s_ref[...]
    m = jnp.max(srow, axis=1, keepdims=True)
    e = jnp.exp(srow - m)
    ssm = e / jnp.sum(e, axis=1, keepdims=True)
    srep = jnp.dot(ssm, rep_ref[...], preferred_element_type=jnp.float32)

    theta = ang_ref[...] * srep
    c = jnp.cos(theta)
    s = jnp.sin(theta)
    rotm = rotm_ref[...]
    q_out[...] = q * c + jnp.dot(q, rotm, preferred_element_type=jnp.float32) * s
    k_out[...] = k * c + jnp.dot(k, rotm, preferred_element_type=jnp.float32) * s
    v_out[...] = v


def _dense_call(x, ang, wqt, wkt, wvt, bq, bk, bv, s, rep, rotm):
    n = x.shape[0]
    grid = (n // RB,)
    full = lambda shp: pl.BlockSpec(shp, lambda i: (0,) * len(shp))
    return pl.pallas_call(
        _dense_body,
        grid=grid,
        in_specs=[
            pl.BlockSpec((RB, F), lambda i: (i, 0)),
            full((F, HD)), full((F, HD)), full((F, HD)),
            full((1, HD)), full((1, HD)), full((1, HD)),
            pl.BlockSpec((RB, NA), lambda i: (i, 0)),
            full((NA, HD // 2)), full((HD // 2, HD)), full((HD, HD)),
        ],
        out_specs=[pl.BlockSpec((RB, HD), lambda i: (i, 0))] * 3,
        out_shape=[jax.ShapeDtypeStruct((n, HD), jnp.float32)] * 3,
    )(x, wqt, wkt, wvt, bq, bk, bv, ang, s, rep, rotm)


def _chunks(total, step):
    out = []
    r = 0
    while r < total:
        n = min(step, total - r)
        out.append((r, n))
        r += n
    return out


def _sc_body(k_hbm, q_hbm, v_hbm, src_hbm, dst_hbm,
             acc_out, den_out,
             sidx, didx, didx16, kbuf, qbuf, dbuf, wbuf, sh_acc, sh_den,
             sem0, sem1, sem2, sem3):
    cid = lax.axis_index("c")
    sid = lax.axis_index("s")
    wid = sid * NC + cid
    base = wid * CHUNK
    tb = sid * ROWS_PER_TILE

    @pl.loop(0, BLK)
    def _zero_bufs(i):
        wbuf[i] = jnp.zeros((16,), jnp.float32)
        for c in range(H):
            kbuf[i, pl.ds(c * 16, 16)] = jnp.zeros((16,), jnp.float32)
            dbuf[i, pl.ds(c * 16, 16)] = jnp.zeros((16,), jnp.float32)

    for r0, nr in _chunks(ROWS_PER_TILE, BLK):
        pltpu.sync_copy(kbuf.at[pl.ds(0, nr)], sh_acc.at[pl.ds(tb + r0, nr)])

    @pl.when(sid == 0)
    def _zero_den():
        for r0, nr in _chunks(NDEN, BLK):
            pltpu.sync_copy(dbuf.at[pl.ds(0, nr)], sh_den.at[pl.ds(r0, nr)])

    plsc.subcore_barrier()

    @pl.loop(0, NSUP)
    def _super(si):
        eb0 = base + si * (SB * BLK)
        cps = [pltpu.async_copy(src_hbm.at[pl.ds(eb0 + sb * BLK, BLK)],
                                sidx.at[sb], sem3) for sb in range(SB)]
        cps += [pltpu.async_copy(dst_hbm.at[pl.ds(eb0 + sb * BLK, BLK)],
                                 didx.at[sb], sem3) for sb in range(SB)]
        for cp in cps:
            cp.wait()

        for sb in range(SB):
            src_ids = sidx.at[sb]
            cp_k = pltpu.async_copy(k_hbm.at[src_ids], kbuf, sem0)
            cp_q = pltpu.async_copy(q_hbm.at[didx.at[sb]], qbuf, sem1)
            cp_k.wait()
            cp_q.wait()

            @pl.loop(0, NGRP)
            def _group(g):
                j0 = g * 16
                eidx = j0 + lax.iota(jnp.int32, 16)
                inv_sqrt_d = 1.0 / (D ** 0.5)
                for h in range(H):
                    col0 = h * D
                    acc = jnp.zeros((16,), jnp.float32)
                    for dd in range(D):
                        cvec = jnp.full((16,), col0 + dd, jnp.int32)
                        kc = plsc.load_gather(kbuf, [eidx, cvec])
                        qc = plsc.load_gather(qbuf, [eidx, cvec])
                        acc = acc + kc * qc
                    sc = jnp.clip(acc * inv_sqrt_d, -CLAMP, CLAMP)
                    wh = jnp.exp(sc)
                    plsc.store_scatter(
                        wbuf, [eidx, jnp.full((16,), h, jnp.int32)], wh)

            cp_v = pltpu.async_copy(v_hbm.at[src_ids], kbuf, sem2)

            @pl.loop(0, NGRP)
            def _fill(g):
                j0 = g * 16
                eidx = j0 + lax.iota(jnp.int32, 16)
                dvec = didx[sb, pl.ds(j0, 16)]
                didx16[sb, pl.ds(j0, 16)] = lax.shift_right_logical(dvec, 4)
                colbase = lax.shift_left(dvec & 15, 3)
                for h in range(H):
                    wh = plsc.load_gather(
                        wbuf, [eidx, jnp.full((16,), h, jnp.int32)])
                    plsc.store_scatter(dbuf, [eidx, colbase + h], wh)

            cp_v.wait()

            @pl.loop(0, BLK)
            def _scale(row):
                rowv = jnp.broadcast_to(row, (16,)).astype(jnp.int32)
                for h in range(H):
                    wb = plsc.load_gather(
                        wbuf, [rowv, jnp.full((16,), h, jnp.int32)])
                    vrow = kbuf[row, pl.ds(h * D, 16)]
                    kbuf[row, pl.ds(h * D, 16)] = vrow * wb

            cp_n = pltpu.async_copy(kbuf, sh_acc.at[didx.at[sb]], sem0, add=True)
            cp_d = pltpu.async_copy(dbuf, sh_den.at[didx16.at[sb]], sem1, add=True)
            cp_n.wait()
            cp_d.wait()

            @pl.loop(0, BLK)
            def _zero_d(i):
                for c in range(H):
                    dbuf[i, pl.ds(c * 16, 16)] = jnp.zeros((16,), jnp.float32)

    plsc.subcore_barrier()

    for r0, nr in _chunks(ROWS_PER_TILE, BLK):
        pltpu.sync_copy(sh_acc.at[pl.ds(tb + r0, nr)], kbuf.at[pl.ds(0, nr)])
        pltpu.sync_copy(kbuf.at[pl.ds(0, nr)],
                        acc_out.at[cid, pl.ds(tb + r0, nr)])

    @pl.when(sid == 0)
    def _dump_den():
        for r0, nr in _chunks(NDEN, BLK):
            pltpu.sync_copy(sh_den.at[pl.ds(r0, nr)], dbuf.at[pl.ds(0, nr)])
            pltpu.sync_copy(dbuf.at[pl.ds(0, nr)],
                            den_out.at[cid, pl.ds(r0, nr)])


def _sc_call(k_rot, q_rot, v_h, src, dst):
    mesh = plsc.VectorSubcoreMesh(core_axis_name="c", subcore_axis_name="s")
    kern = functools.partial(
        pl.kernel,
        out_type=[
            jax.ShapeDtypeStruct((NC, NPAD, HD), jnp.float32),
            jax.ShapeDtypeStruct((NC, NDEN, HD), jnp.float32),
        ],
        mesh=mesh,
        compiler_params=pltpu.CompilerParams(needs_layout_passes=False),
        scratch_types=[
            pltpu.VMEM((SB, BLK), jnp.int32),
            pltpu.VMEM((SB, BLK), jnp.int32),
            pltpu.VMEM((SB, BLK), jnp.int32),
            pltpu.VMEM((BLK, HD), jnp.float32),
            pltpu.VMEM((BLK, HD), jnp.float32),
            pltpu.VMEM((BLK, HD), jnp.float32),
            pltpu.VMEM((BLK, 16), jnp.float32),
            pltpu.VMEM_SHARED((NPAD, HD), jnp.float32),
            pltpu.VMEM_SHARED((NDEN, HD), jnp.float32),
            pltpu.SemaphoreType.DMA,
            pltpu.SemaphoreType.DMA,
            pltpu.SemaphoreType.DMA,
            pltpu.SemaphoreType.DMA,
        ],
    )(_sc_body)
    return kern(k_rot, q_rot, v_h, src, dst)


def _combine_body(acc_ref, den_ref, sel_ref, out_ref):
    nsum = acc_ref[0] + acc_ref[1]
    den8 = den_ref[0] + den_ref[1]
    drep = jnp.dot(den8, sel_ref[...], preferred_element_type=jnp.float32)
    out_ref[...] = nsum / (drep + 1e-16)


def _combine_call(acc, den, sel):
    grid = (N_NODES // RB,)
    return pl.pallas_call(
        _combine_body,
        grid=grid,
        in_specs=[
            pl.BlockSpec((NC, RB, HD), lambda i: (0, i, 0)),
            pl.BlockSpec((NC, RB, H), lambda i: (0, i, 0)),
            pl.BlockSpec((H, HD), lambda i: (0, 0)),
        ],
        out_specs=pl.BlockSpec((RB, HD), lambda i: (i, 0)),
        out_shape=jax.ShapeDtypeStruct((N_NODES, HD), jnp.float32),
    )(acc, den, sel)


def kernel(x, edge_index, node_rotation_angles, Wq, bq, Wk, bk, Wv, bv, S):
    n = x.shape[0]

    rotm = np.zeros((HD, HD), np.float32)
    for h in range(H):
        b = h * D
        for j in range(D // 2):
            rotm[b + 2 * j + 1, b + j] = -1.0
            rotm[b + 2 * j, b + D // 2 + j] = 1.0
    rep = np.zeros((HD // 2, HD), np.float32)
    for i in range(HD // 2):
        rep[i, 2 * i] = 1.0
        rep[i, 2 * i + 1] = 1.0
    sel = np.zeros((H, HD), np.float32)
    for h in range(H):
        sel[h, h * D:(h + 1) * D] = 1.0
    rotm = jnp.asarray(rotm)
    rep = jnp.asarray(rep)
    sel = jnp.asarray(sel)

    q_rot, k_rot, v_h = _dense_call(
        x, node_rotation_angles,
        Wq.T, Wk.T, Wv.T,
        bq.reshape(1, HD), bk.reshape(1, HD), bv.reshape(1, HD),
        S, rep, rotm)

    src = edge_index[0]
    dst = edge_index[1]
    acc, den = _sc_call(k_rot, q_rot, v_h, src, dst)

    den_nodes = den.reshape(NC, NPAD, H)
    wv = _combine_call(acc, den_nodes, sel)
    return wv.reshape(n, H, D)
